# baseline (speedup 1.0000x reference)
import functools

import jax
import jax.numpy as jnp
from jax import lax
from jax.experimental import pallas as pl
from jax.experimental.pallas import tpu as pltpu
from jax.experimental.pallas import tpu_sc as plsc

N = 10000
NPAD = 10240
DUMMY = 10000
E = 320000
NC, NS = 2, 16
NW = NC * NS
CHUNK = 128
CWA = 144
CWB = 16
CWM = max(CWA, CWB)
DCW = 80
EPAD = NS * (CWA + CWB) * CHUNK
RPT = NPAD // NS
RB = 1024

_mesh = plsc.VectorSubcoreMesh(core_axis_name="c", subcore_axis_name="s")


def _deg_body(dst_hbm, z1_hbm, deg_out, didx, ones_ref, acc):
    c = lax.axis_index("c")
    s = lax.axis_index("s")
    wid = c * NS + s
    pltpu.sync_copy(z1_hbm.at[pl.ds(s * RPT, RPT)], acc.at[pl.ds(s * RPT, RPT)])
    for i in range(CHUNK // 16):
        ones_ref[pl.ds(i * 16, 16)] = jnp.full((16,), 1.0, jnp.float32)
    pltpu.sync_copy(dst_hbm.at[wid], didx)
    plsc.subcore_barrier()

    def step(j, carry):
        pltpu.sync_copy(ones_ref, acc.at[didx.at[j]], add=True)
        return carry

    lax.fori_loop(0, DCW, step, 0, unroll=False)
    plsc.subcore_barrier()
    pltpu.sync_copy(acc.at[pl.ds(s * RPT, RPT)],
                    deg_out.at[c, pl.ds(s * RPT, RPT)])


_deg_kernel = functools.partial(
    pl.kernel,
    out_type=jax.ShapeDtypeStruct((NC, NPAD), jnp.float32),
    mesh=_mesh,
    scratch_types=[
        pltpu.VMEM((DCW, CHUNK), jnp.int32),
        pltpu.VMEM((CHUNK,), jnp.float32),
        pltpu.VMEM_SHARED((NPAD,), jnp.float32),
    ],
)(_deg_body)


def _make_spmm(C):
    def body(pidx_hbm, g_hbm, outf_hbm, outb_hbm,
             pidx, sring, dring, rows, acc):
        c = lax.axis_index("c")
        s = lax.axis_index("s")
        wid = c * NS + s

        def zrow(r, carry):
            for u in range(C // 16):
                rows[r, pl.ds(u * 16, 16)] = jnp.zeros((16,), jnp.float32)
            return carry

        lax.fori_loop(0, CHUNK, zrow, 0, unroll=False)
        for q in range(RPT // CHUNK):
            pltpu.sync_copy(
                rows, acc.at[pl.ds(s * RPT + q * CHUNK, CHUNK)])
        pltpu.sync_copy(pidx_hbm.at[wid], pidx)
        plsc.subcore_barrier()

        def step(j, carry):
            for v in range(CHUNK // 16):
                pk = pidx[j, pl.ds(v * 16, 16)]
                sring[pl.ds(v * 16, 16)] = pk & 0x3FFF
                dring[pl.ds(v * 16, 16)] = pk >> 14
            pltpu.sync_copy(g_hbm.at[sring], rows)
            pltpu.sync_copy(rows, acc.at[dring], add=True)
            return carry

        nch = jnp.where(c == 0, CWA, CWB)
        lax.fori_loop(0, nch, step, 0, unroll=False)
        plsc.subcore_barrier()

        @pl.when(c == 0)
        def _wf():
            pltpu.sync_copy(acc.at[pl.ds(s * RPT, RPT)],
                            outf_hbm.at[pl.ds(s * RPT, RPT)])

        @pl.when(c == 1)
        def _wb():
            pltpu.sync_copy(acc.at[pl.ds(s * RPT, RPT)],
                            outb_hbm.at[pl.ds(s * RPT, RPT)])

    return pl.kernel(
        body,
        out_type=[
            jax.ShapeDtypeStruct((NPAD, C), jnp.float32),
            jax.ShapeDtypeStruct((NPAD, C), jnp.float32),
        ],
        mesh=_mesh,
        compiler_params=pltpu.CompilerParams(use_tc_tiling_on_sc=(C == 128)),
        scratch_types=[
            pltpu.VMEM((CWM, CHUNK), jnp.int32),
            pltpu.VMEM((CHUNK,), jnp.int32),
            pltpu.VMEM((CHUNK,), jnp.int32),
            pltpu.VMEM((CHUNK, C), jnp.float32),
            pltpu.VMEM_SHARED((NPAD, C), jnp.float32),
        ],
    )


_spmm128 = _make_spmm(128)
_spmm64 = _make_spmm(64)


def _tc1_body(degp_ref, x_ref, w_ref, g_ref, dis_ref):
    deg = degp_ref[0] + degp_ref[1] + 1.0
    dis = lax.rsqrt(deg)
    dis_ref[...] = dis
    g_ref[...] = jnp.dot(x_ref[...] * dis, w_ref[...],
                         preferred_element_type=jnp.float32)


def _tc_first(degp, x, w):
    grid = NPAD // RB
    return pl.pallas_call(
        _tc1_body,
        grid=(grid,),
        in_specs=[
            pl.BlockSpec((2, RB, 1), lambda i: (0, i, 0)),
            pl.BlockSpec((RB, 128), lambda i: (i, 0)),
            pl.BlockSpec((128, 128), lambda i: (0, 0)),
        ],
        out_specs=[
            pl.BlockSpec((RB, 128), lambda i: (i, 0)),
            pl.BlockSpec((RB, 1), lambda i: (i, 0)),
        ],
        out_shape=[
            jax.ShapeDtypeStruct((NPAD, 128), jnp.float32),
            jax.ShapeDtypeStruct((NPAD, 1), jnp.float32),
        ],
    )(degp, x, w)


def _tc_mid_body(pf_ref, pb_ref, g_ref, dis_ref, b_ref, w_ref, o_ref):
    dis = dis_ref[...]
    p = pf_ref[...] + pb_ref[...]
    pre = (p + g_ref[...]) * dis + b_ref[...]
    h = jnp.maximum(pre, 0.0)
    o_ref[...] = jnp.dot(h * dis, w_ref[...],
                         preferred_element_type=jnp.float32)


def _tc_mid(pf, pb, g, dis, b, w, cin, cout):
    grid = NPAD // RB
    return pl.pallas_call(
        _tc_mid_body,
        grid=(grid,),
        in_specs=[
            pl.BlockSpec((RB, cin), lambda i: (i, 0)),
            pl.BlockSpec((RB, cin), lambda i: (i, 0)),
            pl.BlockSpec((RB, cin), lambda i: (i, 0)),
            pl.BlockSpec((RB, 1), lambda i: (i, 0)),
            pl.BlockSpec((1, cin), lambda i: (0, 0)),
            pl.BlockSpec((cin, cout), lambda i: (0, 0)),
        ],
        out_specs=pl.BlockSpec((RB, cout), lambda i: (i, 0)),
        out_shape=jax.ShapeDtypeStruct((NPAD, cout), jnp.float32),
    )(pf, pb, g, dis, b, w)


def _tc_last_body(pf_ref, pb_ref, g_ref, dis_ref, b_ref, o_ref):
    p = pf_ref[...] + pb_ref[...]
    z = (p + g_ref[...]) * dis_ref[...] + b_ref[...]
    m = jnp.max(z, axis=1, keepdims=True)
    e = jnp.exp(z - m)
    lse = jnp.log(jnp.sum(e, axis=1, keepdims=True))
    o_ref[...] = z - m - lse


def _tc_last(pf, pb, g, dis, b):
    grid = NPAD // RB
    return pl.pallas_call(
        _tc_last_body,
        grid=(grid,),
        in_specs=[
            pl.BlockSpec((RB, 64), lambda i: (i, 0)),
            pl.BlockSpec((RB, 64), lambda i: (i, 0)),
            pl.BlockSpec((RB, 64), lambda i: (i, 0)),
            pl.BlockSpec((RB, 1), lambda i: (i, 0)),
            pl.BlockSpec((1, 64), lambda i: (0, 0)),
        ],
        out_specs=pl.BlockSpec((RB, 64), lambda i: (i, 0)),
        out_shape=jax.ShapeDtypeStruct((NPAD, 64), jnp.float32),
    )(pf, pb, g, dis, b)


def kernel(x, edge_index, omega, partition, W1, b1, W2, b2, W3, b3):
    src = edge_index[0]
    dst = edge_index[1]
    pad = EPAD - E
    pad_dst = (N + 16 + (jnp.arange(pad, dtype=jnp.int32) % (NPAD - N - 16)))
    src_f = jnp.concatenate([src, jnp.zeros((pad,), jnp.int32)])
    dst_f = jnp.concatenate([dst, pad_dst])
    pk_f = src_f | (dst_f << 14)
    cut = NS * CWA * CHUNK
    fill = (N + 16) << 14
    pa = jnp.pad(pk_f[:cut].reshape(NS, CWA, CHUNK),
                 ((0, 0), (0, CWM - CWA), (0, 0)), constant_values=fill)
    pb = jnp.pad(pk_f[cut:].reshape(NS, CWB, CHUNK),
                 ((0, 0), (0, CWM - CWB), (0, 0)), constant_values=fill)
    packed = jnp.concatenate([pa, pb], axis=0)
    dst_g = dst_f.reshape(NW, DCW, CHUNK)
    xp = jnp.pad(x, ((0, NPAD - N), (0, 0)))
    z1 = jnp.zeros((NPAD,), jnp.float32)

    degp = _deg_kernel(dst_g, z1)
    degp3 = degp.reshape(NC, NPAD, 1)

    g1, dis = _tc_first(degp3, xp, W1)
    p1f, p1b = _spmm128(packed, g1)
    g2 = _tc_mid(p1f, p1b, g1, dis, b1.reshape(1, 128), W2, 128, 128)
    p2f, p2b = _spmm128(packed, g2)
    g3 = _tc_mid(p2f, p2b, g2, dis, b2.reshape(1, 128), W3, 128, 64)
    p3f, p3b = _spmm64(packed, g3)
    out = _tc_last(p3f, p3b, g3, dis, b3.reshape(1, 64))
    return out[:N]


# speedup vs baseline: 1.0162x; 1.0162x over previous
import functools

import jax
import jax.numpy as jnp
from jax import lax
from jax.experimental import pallas as pl
from jax.experimental.pallas import tpu as pltpu
from jax.experimental.pallas import tpu_sc as plsc

N = 10000
NPAD = 10240
E = 320000
NC, NS = 2, 16
NW = NC * NS
CHUNK = 128
CWA = 140
CWB = 20
CWM = max(CWA, CWB)
DCW = 80
EPAD = NS * (CWA + CWB) * CHUNK
RPT = NPAD // NS
RB = 1024

_mesh = plsc.VectorSubcoreMesh(core_axis_name="c", subcore_axis_name="s")


def _deg_body(dst_hbm, z1_hbm, deg_out, didx, ones_ref, acc):
    c = lax.axis_index("c")
    s = lax.axis_index("s")
    wid = c * NS + s
    pltpu.sync_copy(z1_hbm.at[pl.ds(s * RPT, RPT)], acc.at[pl.ds(s * RPT, RPT)])
    for i in range(CHUNK // 16):
        ones_ref[pl.ds(i * 16, 16)] = jnp.full((16,), 1.0, jnp.float32)
    pltpu.sync_copy(dst_hbm.at[wid], didx)
    plsc.subcore_barrier()

    def step(j, carry):
        pltpu.sync_copy(ones_ref, acc.at[didx.at[j]], add=True)
        return carry

    lax.fori_loop(0, DCW, step, 0, unroll=False)
    plsc.subcore_barrier()
    pltpu.sync_copy(acc.at[pl.ds(s * RPT, RPT)],
                    deg_out.at[c, pl.ds(s * RPT, RPT)])


_deg_kernel = functools.partial(
    pl.kernel,
    out_type=jax.ShapeDtypeStruct((NC, NPAD), jnp.float32),
    mesh=_mesh,
    scratch_types=[
        pltpu.VMEM((DCW, CHUNK), jnp.int32),
        pltpu.VMEM((CHUNK,), jnp.float32),
        pltpu.VMEM_SHARED((NPAD,), jnp.float32),
    ],
)(_deg_body)


def _make_spmm(C):
    def body(pidx_hbm, g_hbm, outf_hbm, outb_hbm,
             pidx, sring, dring, rows, acc):
        c = lax.axis_index("c")
        s = lax.axis_index("s")
        wid = c * NS + s

        def zrow(r, carry):
            for u in range(C // 16):
                rows[r, pl.ds(u * 16, 16)] = jnp.zeros((16,), jnp.float32)
            return carry

        lax.fori_loop(0, CHUNK, zrow, 0, unroll=False)
        for q in range(RPT // CHUNK):
            pltpu.sync_copy(
                rows, acc.at[pl.ds(s * RPT + q * CHUNK, CHUNK)])
        pltpu.sync_copy(pidx_hbm.at[wid], pidx)
        plsc.subcore_barrier()

        def step(j, carry):
            for v in range(CHUNK // 16):
                pk = pidx[j, pl.ds(v * 16, 16)]
                sring[pl.ds(v * 16, 16)] = pk & 0x3FFF
                dring[pl.ds(v * 16, 16)] = pk >> 14
            pltpu.sync_copy(g_hbm.at[sring], rows)
            pltpu.sync_copy(rows, acc.at[dring], add=True)
            return carry

        nch = jnp.where(c == 0, CWA, CWB)
        lax.fori_loop(0, nch, step, 0, unroll=False)
        plsc.subcore_barrier()

        @pl.when(c == 0)
        def _wf():
            pltpu.sync_copy(acc.at[pl.ds(s * RPT, RPT)],
                            outf_hbm.at[pl.ds(s * RPT, RPT)])

        @pl.when(c == 1)
        def _wb():
            pltpu.sync_copy(acc.at[pl.ds(s * RPT, RPT)],
                            outb_hbm.at[pl.ds(s * RPT, RPT)])

    return pl.kernel(
        body,
        out_type=[
            jax.ShapeDtypeStruct((NPAD, C), jnp.float32),
            jax.ShapeDtypeStruct((NPAD, C), jnp.float32),
        ],
        mesh=_mesh,
        compiler_params=pltpu.CompilerParams(use_tc_tiling_on_sc=(C == 128)),
        scratch_types=[
            pltpu.VMEM((CWM, CHUNK), jnp.int32),
            pltpu.VMEM((CHUNK,), jnp.int32),
            pltpu.VMEM((CHUNK,), jnp.int32),
            pltpu.VMEM((CHUNK, C), jnp.float32),
            pltpu.VMEM_SHARED((NPAD, C), jnp.float32),
        ],
    )


_spmm128 = _make_spmm(128)
_spmm64 = _make_spmm(64)


def _tc1_body(degp_ref, x_ref, w_ref, g_ref, dis_ref):
    deg = degp_ref[0] + degp_ref[1] + 1.0
    dis = lax.rsqrt(deg)
    dis_ref[...] = dis
    g_ref[...] = jnp.dot(x_ref[...] * dis, w_ref[...],
                         preferred_element_type=jnp.float32)


def _tc_first(degp, x, w):
    grid = NPAD // RB
    return pl.pallas_call(
        _tc1_body,
        grid=(grid,),
        in_specs=[
            pl.BlockSpec((2, RB, 1), lambda i: (0, i, 0)),
            pl.BlockSpec((RB, 128), lambda i: (i, 0)),
            pl.BlockSpec((128, 128), lambda i: (0, 0)),
        ],
        out_specs=[
            pl.BlockSpec((RB, 128), lambda i: (i, 0)),
            pl.BlockSpec((RB, 1), lambda i: (i, 0)),
        ],
        out_shape=[
            jax.ShapeDtypeStruct((NPAD, 128), jnp.float32),
            jax.ShapeDtypeStruct((NPAD, 1), jnp.float32),
        ],
    )(degp, x, w)


def _tc_mid_body(pf_ref, pb_ref, g_ref, dis_ref, b_ref, w_ref, o_ref):
    dis = dis_ref[...]
    p = pf_ref[...] + pb_ref[...]
    pre = (p + g_ref[...]) * dis + b_ref[...]
    h = jnp.maximum(pre, 0.0)
    o_ref[...] = jnp.dot(h * dis, w_ref[...],
                         preferred_element_type=jnp.float32)


def _tc_mid(pf, pb, g, dis, b, w, cin, cout):
    grid = NPAD // RB
    return pl.pallas_call(
        _tc_mid_body,
        grid=(grid,),
        in_specs=[
            pl.BlockSpec((RB, cin), lambda i: (i, 0)),
            pl.BlockSpec((RB, cin), lambda i: (i, 0)),
            pl.BlockSpec((RB, cin), lambda i: (i, 0)),
            pl.BlockSpec((RB, 1), lambda i: (i, 0)),
            pl.BlockSpec((1, cin), lambda i: (0, 0)),
            pl.BlockSpec((cin, cout), lambda i: (0, 0)),
        ],
        out_specs=pl.BlockSpec((RB, cout), lambda i: (i, 0)),
        out_shape=jax.ShapeDtypeStruct((NPAD, cout), jnp.float32),
    )(pf, pb, g, dis, b, w)


def _tc_last_body(pf_ref, pb_ref, g_ref, dis_ref, b_ref, o_ref):
    p = pf_ref[...] + pb_ref[...]
    z = (p + g_ref[...]) * dis_ref[...] + b_ref[...]
    m = jnp.max(z, axis=1, keepdims=True)
    e = jnp.exp(z - m)
    lse = jnp.log(jnp.sum(e, axis=1, keepdims=True))
    o_ref[...] = z - m - lse


def _tc_last(pf, pb, g, dis, b):
    grid = NPAD // RB
    return pl.pallas_call(
        _tc_last_body,
        grid=(grid,),
        in_specs=[
            pl.BlockSpec((RB, 64), lambda i: (i, 0)),
            pl.BlockSpec((RB, 64), lambda i: (i, 0)),
            pl.BlockSpec((RB, 64), lambda i: (i, 0)),
            pl.BlockSpec((RB, 1), lambda i: (i, 0)),
            pl.BlockSpec((1, 64), lambda i: (0, 0)),
        ],
        out_specs=pl.BlockSpec((RB, 64), lambda i: (i, 0)),
        out_shape=jax.ShapeDtypeStruct((NPAD, 64), jnp.float32),
    )(pf, pb, g, dis, b)


def kernel(x, edge_index, omega, partition, W1, b1, W2, b2, W3, b3):
    src = edge_index[0]
    dst = edge_index[1]
    pad = EPAD - E
    pad_dst = (N + 16 + (jnp.arange(pad, dtype=jnp.int32) % (NPAD - N - 16)))
    src_f = jnp.concatenate([src, jnp.zeros((pad,), jnp.int32)])
    dst_f = jnp.concatenate([dst, pad_dst])
    pk_f = src_f | (dst_f << 14)
    cut = NS * CWA * CHUNK
    fill = (N + 16) << 14
    pa = jnp.pad(pk_f[:cut].reshape(NS, CWA, CHUNK),
                 ((0, 0), (0, CWM - CWA), (0, 0)), constant_values=fill)
    pb = jnp.pad(pk_f[cut:].reshape(NS, CWB, CHUNK),
                 ((0, 0), (0, CWM - CWB), (0, 0)), constant_values=fill)
    packed = jnp.concatenate([pa, pb], axis=0)
    dst_g = dst_f.reshape(NW, DCW, CHUNK)
    xp = jnp.pad(x, ((0, NPAD - N), (0, 0)))
    z1 = jnp.zeros((NPAD,), jnp.float32)

    degp = _deg_kernel(dst_g, z1)
    degp3 = degp.reshape(NC, NPAD, 1)

    g1, dis = _tc_first(degp3, xp, W1)
    p1f, p1b = _spmm128(packed, g1)
    g2 = _tc_mid(p1f, p1b, g1, dis, b1.reshape(1, 128), W2, 128, 128)
    p2f, p2b = _spmm128(packed, g2)
    g3 = _tc_mid(p2f, p2b, g2, dis, b2.reshape(1, 128), W3, 128, 64)
    p3f, p3b = _spmm64(packed, g3)
    out = _tc_last(p3f, p3b, g3, dis, b3.reshape(1, 64))
    return out[:N]
